# Initial kernel scaffold; baseline (speedup 1.0000x reference)
import jax
import jax.numpy as jnp
from jax import lax
from jax.experimental import pallas as pl
from jax.experimental.pallas import tpu as pltpu
from jax.experimental.pallas import tpu_sc as plsc

N, C, H, W = 6, 96, 224, 224
HW = H * W
V = 100000
CONF = 32
OUT = 128

NC, NS = 2, 16
NW = NC * NS
CPW = C // NW

CHUNK = 12544
NCHUNK = HW // CHUNK
UNROLL = 8
GROUPS = CHUNK // (16 * UNROLL)


def _sc_body(feats_hbm, ids_hbm, sums_hbm, cnts_hbm, acc, idbuf, valbuf):
    wid = lax.axis_index("s") * NC + lax.axis_index("c")

    zeros16 = jnp.zeros((16,), jnp.float32)
    ones16 = jnp.ones((16,), jnp.float32)

    def zero_acc():
        def zb(i, carry):
            acc[pl.ds(i * 16, 16)] = zeros16
            return carry
        lax.fori_loop(0, V // 16, zb, 0)

    def scatter_chunk(use_vals):
        def gb(g, carry):
            base = g * (16 * UNROLL)
            for u in range(UNROLL):
                off = base + u * 16
                idx = idbuf[pl.ds(off, 16)]
                v = valbuf[pl.ds(off, 16)] if use_vals else ones16
                plsc.addupdate_scatter(acc, [idx], v)
            return carry
        lax.fori_loop(0, GROUPS, gb, 0)

    @pl.when(wid < N)
    def _():
        zero_acc()

        def chunk_body(q, carry):
            off = q * CHUNK
            pltpu.sync_copy(ids_hbm.at[wid, pl.ds(off, CHUNK)], idbuf)
            scatter_chunk(False)
            return carry
        lax.fori_loop(0, NCHUNK, chunk_body, 0)
        pltpu.sync_copy(acc, cnts_hbm.at[wid])

    for k in range(CPW):
        ch = wid * CPW + k
        zero_acc()

        def plane_body(n, carry, ch=ch):
            row = n * C + ch

            def chunk_body(q, carry2):
                off = q * CHUNK
                pltpu.sync_copy(ids_hbm.at[n, pl.ds(off, CHUNK)], idbuf)
                pltpu.sync_copy(feats_hbm.at[row, pl.ds(off, CHUNK)], valbuf)
                scatter_chunk(True)
                return carry2
            lax.fori_loop(0, NCHUNK, chunk_body, 0)
            return carry
        lax.fori_loop(0, N, plane_body, 0)
        pltpu.sync_copy(acc, sums_hbm.at[ch])


_sc_scatter = pl.kernel(
    _sc_body,
    out_type=[
        jax.ShapeDtypeStruct((C, V), jnp.float32),
        jax.ShapeDtypeStruct((N, V), jnp.float32),
    ],
    mesh=plsc.VectorSubcoreMesh(
        core_axis_name="c", subcore_axis_name="s",
        num_cores=NC, num_subcores=NS,
    ),
    scratch_types=[
        pltpu.VMEM((V,), jnp.float32),
        pltpu.VMEM((CHUNK,), jnp.int32),
        pltpu.VMEM((CHUNK,), jnp.float32),
    ],
)


VB = 2000


def _tc_body(sums_ref, cnts_ref, conf_ref, w_ref, b_ref, out_ref):
    s = sums_ref[...]
    cnt = jnp.sum(cnts_ref[...], axis=0, keepdims=True)
    sv = s / jnp.maximum(cnt, 1.0)
    w = w_ref[...]
    w1 = w[:, :C]
    w2 = w[:, C:]
    a = lax.dot_general(sv, w1, (((0,), (1,)), ((), ())),
                        preferred_element_type=jnp.float32)
    b2 = lax.dot_general(conf_ref[...], w2, (((1,), (1,)), ((), ())),
                         preferred_element_type=jnp.float32)
    out_ref[...] = a + b2 + b_ref[...]


_tc_mix = pl.pallas_call(
    _tc_body,
    grid=(V // VB,),
    in_specs=[
        pl.BlockSpec((C, VB), lambda i: (0, i)),
        pl.BlockSpec((N, VB), lambda i: (0, i)),
        pl.BlockSpec((VB, CONF), lambda i: (i, 0)),
        pl.BlockSpec((OUT, C + CONF), lambda i: (0, 0)),
        pl.BlockSpec((1, OUT), lambda i: (0, 0)),
    ],
    out_specs=pl.BlockSpec((VB, OUT), lambda i: (i, 0)),
    out_shape=jax.ShapeDtypeStruct((V, OUT), jnp.float32),
)


def kernel(camera_pose, padded_intrinsics, padded_img_features, depths,
           padding_confidence, out_voxel_ids, W_mix, b_mix):
    feats = padded_img_features.reshape(N * C, HW)
    ids = out_voxel_ids.reshape(N, HW).astype(jnp.int32)
    conf = padding_confidence.reshape(V, CONF)
    sums, cnts = _sc_scatter(feats, ids)
    out = _tc_mix(sums, cnts, conf, W_mix, b_mix.reshape(1, OUT))
    return out.reshape(1, V, OUT)


# speedup vs baseline: 1.8341x; 1.8341x over previous
import jax
import jax.numpy as jnp
from jax import lax
from jax.experimental import pallas as pl
from jax.experimental.pallas import tpu as pltpu
from jax.experimental.pallas import tpu_sc as plsc

N, C, H, W = 6, 96, 224, 224
HW = H * W
V = 100000
CONF = 32
OUT = 128

NC, NS = 2, 16
NW = NC * NS
CPW = C // NW

CHUNK = 12544
NCHUNK = HW // CHUNK
UNROLL = 8
GROUPS = CHUNK // (16 * UNROLL)


def _sc_body(feats_hbm, ids_hbm, sums_hbm, cnts_hbm, acc, idbuf, valbuf):
    wid = lax.axis_index("s") * NC + lax.axis_index("c")

    zeros16 = jnp.zeros((16,), jnp.float32)
    ones16 = jnp.ones((16,), jnp.float32)

    def zero_acc():
        def zb(i, carry):
            acc[pl.ds(i * 16, 16)] = zeros16
            return carry
        lax.fori_loop(0, V // 16, zb, 0)

    def scatter_chunk(use_vals):
        def gb(g, carry):
            base = g * (16 * UNROLL)
            for u in range(UNROLL):
                off = base + u * 16
                idx = idbuf[pl.ds(off, 16)]
                v = valbuf[pl.ds(off, 16)] if use_vals else ones16
                plsc.addupdate_scatter(acc, [idx], v)
            return carry
        lax.fori_loop(0, GROUPS, gb, 0)

    @pl.when(wid < N)
    def _():
        zero_acc()

        def chunk_body(q, carry):
            off = q * CHUNK
            pltpu.sync_copy(ids_hbm.at[wid, pl.ds(off, CHUNK)], idbuf)
            scatter_chunk(False)
            return carry
        lax.fori_loop(0, NCHUNK, chunk_body, 0)
        pltpu.sync_copy(acc, cnts_hbm.at[wid])

    for k in range(CPW):
        ch = wid * CPW + k
        zero_acc()

        def plane_body(n, carry, ch=ch):
            row = n * C + ch

            def chunk_body(q, carry2):
                off = q * CHUNK
                pltpu.sync_copy(ids_hbm.at[n, pl.ds(off, CHUNK)], idbuf)
                pltpu.sync_copy(feats_hbm.at[row, pl.ds(off, CHUNK)], valbuf)
                scatter_chunk(True)
                return carry2
            lax.fori_loop(0, NCHUNK, chunk_body, 0)
            return carry
        lax.fori_loop(0, N, plane_body, 0)
        pltpu.sync_copy(acc, sums_hbm.at[ch])


_sc_scatter = pl.kernel(
    _sc_body,
    out_type=[
        jax.ShapeDtypeStruct((C, V), jnp.float32),
        jax.ShapeDtypeStruct((N, V), jnp.float32),
    ],
    mesh=plsc.VectorSubcoreMesh(
        core_axis_name="c", subcore_axis_name="s",
        num_cores=NC, num_subcores=NS,
    ),
    scratch_types=[
        pltpu.VMEM((V,), jnp.float32),
        pltpu.VMEM((CHUNK,), jnp.int32),
        pltpu.VMEM((CHUNK,), jnp.float32),
    ],
    compiler_params=pltpu.CompilerParams(
        use_tc_tiling_on_sc=False, needs_layout_passes=False),
)


VB = 2048


def _tc_body(sums_ref, cnts_ref, conf_ref, w_ref, b_ref, out_ref):
    s = sums_ref[...]
    cnt = jnp.sum(cnts_ref[...], axis=0, keepdims=True)
    sv = s / jnp.maximum(cnt, 1.0)
    w = w_ref[...]
    w1 = w[:, :C]
    w2 = w[:, C:]
    a = lax.dot_general(sv, w1, (((0,), (1,)), ((), ())),
                        preferred_element_type=jnp.float32)
    b2 = lax.dot_general(conf_ref[...], w2, (((1,), (1,)), ((), ())),
                         preferred_element_type=jnp.float32)
    out_ref[...] = a + b2 + b_ref[...]


_tc_mix = pl.pallas_call(
    _tc_body,
    grid=(pl.cdiv(V, VB),),
    in_specs=[
        pl.BlockSpec((C, VB), lambda i: (0, i)),
        pl.BlockSpec((N, VB), lambda i: (0, i)),
        pl.BlockSpec((VB, CONF), lambda i: (i, 0)),
        pl.BlockSpec((OUT, C + CONF), lambda i: (0, 0)),
        pl.BlockSpec((1, OUT), lambda i: (0, 0)),
    ],
    out_specs=pl.BlockSpec((VB, OUT), lambda i: (i, 0)),
    out_shape=jax.ShapeDtypeStruct((V, OUT), jnp.float32),
)


def kernel(camera_pose, padded_intrinsics, padded_img_features, depths,
           padding_confidence, out_voxel_ids, W_mix, b_mix):
    feats = padded_img_features.reshape(N * C, HW)
    ids = out_voxel_ids.reshape(N, HW).astype(jnp.int32)
    conf = padding_confidence.reshape(V, CONF)
    sums, cnts = _sc_scatter(feats, ids)
    out = _tc_mix(sums, cnts, conf, W_mix, b_mix.reshape(1, OUT))
    return out.reshape(1, V, OUT)


# speedup vs baseline: 2.5078x; 1.3674x over previous
import jax
import jax.numpy as jnp
from jax import lax
from jax.experimental import pallas as pl
from jax.experimental.pallas import tpu as pltpu
from jax.experimental.pallas import tpu_sc as plsc

N, C, H, W = 6, 96, 224, 224
HW = H * W
V = 100000
CONF = 32
OUT = 128

NC, NS = 2, 16
NW = NC * NS
CPW = C // NW

CHUNK = 6272
NCHUNK = HW // CHUNK
TOTCH = N * NCHUNK
UNROLL = 8
GROUPS = CHUNK // (16 * UNROLL)


def _sc_body(feats_hbm, ids_hbm, sums_hbm, cnts_hbm, acc, idbuf, valbuf, sems):
    wid = lax.axis_index("s") * NC + lax.axis_index("c")

    zeros16 = jnp.zeros((16,), jnp.float32)
    ones16 = jnp.ones((16,), jnp.float32)

    def zero_acc():
        def zb(i, carry):
            base = i * 128
            for u in range(8):
                acc[pl.ds(base + u * 16, 16)] = zeros16
            return carry
        lax.fori_loop(0, V // 128, zb, 0)
        tail = (V // 128) * 128
        for u in range((V - tail) // 16):
            acc[pl.ds(tail + u * 16, 16)] = zeros16

    def scatter_slot(b, use_vals):
        def gb(g, carry):
            base = g * (16 * UNROLL)
            for u in range(UNROLL):
                off = base + u * 16
                idx = idbuf[b, pl.ds(off, 16)]
                v = valbuf[b, pl.ds(off, 16)] if use_vals else ones16
                plsc.addupdate_scatter(acc, [idx], v)
            return carry
        lax.fori_loop(0, GROUPS, gb, 0)

    def start_ids(b, n, q):
        pltpu.async_copy(ids_hbm.at[n, pl.ds(q * CHUNK, CHUNK)],
                         idbuf.at[b], sems.at[b])

    def start_vals(b, row, q):
        pltpu.async_copy(feats_hbm.at[row, pl.ds(q * CHUNK, CHUNK)],
                         valbuf.at[b], sems.at[b + 2])

    def wait_ids(b):
        pltpu.make_async_copy(ids_hbm.at[0, pl.ds(0, CHUNK)],
                              idbuf.at[b], sems.at[b]).wait()

    def wait_vals(b):
        pltpu.make_async_copy(feats_hbm.at[0, pl.ds(0, CHUNK)],
                              valbuf.at[b], sems.at[b + 2]).wait()

    @pl.when(wid < N)
    def _():
        zero_acc()
        for b in range(2):
            start_ids(b, wid, b)

        def pair(i, carry):
            for b in range(2):
                t = 2 * i + b
                wait_ids(b)
                scatter_slot(b, False)
                nxt = t + 2

                @pl.when(nxt < NCHUNK)
                def _():
                    start_ids(b, wid, nxt)
            return carry
        lax.fori_loop(0, NCHUNK // 2, pair, 0)
        pltpu.sync_copy(acc, cnts_hbm.at[wid])

    for k in range(CPW):
        ch = wid * CPW + k
        zero_acc()
        for b in range(2):
            n0 = b // NCHUNK
            start_ids(b, n0, b % NCHUNK)
            start_vals(b, n0 * C + ch, b % NCHUNK)

        def pair(i, carry, ch=ch):
            for b in range(2):
                t = 2 * i + b
                wait_ids(b)
                wait_vals(b)
                scatter_slot(b, True)
                nxt = t + 2

                @pl.when(nxt < TOTCH)
                def _():
                    n = lax.shift_right_logical(nxt, 3)
                    q = lax.bitwise_and(nxt, NCHUNK - 1)
                    start_ids(b, n, q)
                    start_vals(b, n * C + ch, q)
            return carry
        lax.fori_loop(0, TOTCH // 2, pair, 0)
        pltpu.sync_copy(acc, sums_hbm.at[ch])


_sc_scatter = pl.kernel(
    _sc_body,
    out_type=[
        jax.ShapeDtypeStruct((C, V), jnp.float32),
        jax.ShapeDtypeStruct((N, V), jnp.float32),
    ],
    mesh=plsc.VectorSubcoreMesh(
        core_axis_name="c", subcore_axis_name="s",
        num_cores=NC, num_subcores=NS,
    ),
    scratch_types=[
        pltpu.VMEM((V,), jnp.float32),
        pltpu.VMEM((2, CHUNK), jnp.int32),
        pltpu.VMEM((2, CHUNK), jnp.float32),
        pltpu.SemaphoreType.DMA((4,)),
    ],
    compiler_params=pltpu.CompilerParams(
        use_tc_tiling_on_sc=False, needs_layout_passes=False),
)


VB = 2048


def _tc_body(sums_ref, cnts_ref, conf_ref, w_ref, b_ref, out_ref):
    s = sums_ref[...]
    cnt = jnp.sum(cnts_ref[...], axis=0, keepdims=True)
    sv = s / jnp.maximum(cnt, 1.0)
    w = w_ref[...]
    w1 = w[:, :C]
    w2 = w[:, C:]
    a = lax.dot_general(sv, w1, (((0,), (1,)), ((), ())),
                        preferred_element_type=jnp.float32)
    b2 = lax.dot_general(conf_ref[...], w2, (((1,), (1,)), ((), ())),
                         preferred_element_type=jnp.float32)
    out_ref[...] = a + b2 + b_ref[...]


_tc_mix = pl.pallas_call(
    _tc_body,
    grid=(pl.cdiv(V, VB),),
    in_specs=[
        pl.BlockSpec((C, VB), lambda i: (0, i)),
        pl.BlockSpec((N, VB), lambda i: (0, i)),
        pl.BlockSpec((VB, CONF), lambda i: (i, 0)),
        pl.BlockSpec((OUT, C + CONF), lambda i: (0, 0)),
        pl.BlockSpec((1, OUT), lambda i: (0, 0)),
    ],
    out_specs=pl.BlockSpec((VB, OUT), lambda i: (i, 0)),
    out_shape=jax.ShapeDtypeStruct((V, OUT), jnp.float32),
)


def kernel(camera_pose, padded_intrinsics, padded_img_features, depths,
           padding_confidence, out_voxel_ids, W_mix, b_mix):
    feats = padded_img_features.reshape(N * C, HW)
    ids = out_voxel_ids.reshape(N, HW).astype(jnp.int32)
    conf = padding_confidence.reshape(V, CONF)
    sums, cnts = _sc_scatter(feats, ids)
    out = _tc_mix(sums, cnts, conf, W_mix, b_mix.reshape(1, OUT))
    return out.reshape(1, V, OUT)


# speedup vs baseline: 2.5082x; 1.0001x over previous
import jax
import jax.numpy as jnp
from jax import lax
from jax.experimental import pallas as pl
from jax.experimental.pallas import tpu as pltpu
from jax.experimental.pallas import tpu_sc as plsc

N, C, H, W = 6, 96, 224, 224
HW = H * W
V = 100000
CONF = 32
OUT = 128

NC, NS = 2, 16
NW = NC * NS
CPW = C // NW

CHUNK = 6272
NCHUNK = HW // CHUNK
TOTCH = N * NCHUNK
UNROLL = 8
GROUPS = CHUNK // (16 * UNROLL)


def _sc_body(feats_hbm, ids_hbm, sums_hbm, cnts_hbm, acc, idbuf, valbuf, sems):
    wid = lax.axis_index("s") * NC + lax.axis_index("c")

    zeros16 = jnp.zeros((16,), jnp.float32)
    ones16 = jnp.ones((16,), jnp.float32)

    def zero_acc():
        def zb(i, carry):
            base = i * 128
            for u in range(8):
                acc[pl.ds(base + u * 16, 16)] = zeros16
            return carry
        lax.fori_loop(0, V // 128, zb, 0)
        tail = (V // 128) * 128
        for u in range((V - tail) // 16):
            acc[pl.ds(tail + u * 16, 16)] = zeros16

    def scatter_slot(b, use_vals):
        def gb(g, carry):
            base = g * (16 * UNROLL)
            for u in range(UNROLL):
                off = base + u * 16
                idx = idbuf[b, pl.ds(off, 16)]
                v = valbuf[b, pl.ds(off, 16)] if use_vals else ones16
                plsc.addupdate_scatter(acc, [idx], v)
            return carry
        lax.fori_loop(0, GROUPS, gb, 0)

    def start_ids(b, n, q):
        pltpu.async_copy(ids_hbm.at[pl.ds(n * HW + q * CHUNK, CHUNK)],
                         idbuf.at[b], sems.at[b])

    def start_vals(b, row, q):
        pltpu.async_copy(feats_hbm.at[pl.ds(row * HW + q * CHUNK, CHUNK)],
                         valbuf.at[b], sems.at[b + 2])

    def wait_ids(b):
        pltpu.make_async_copy(ids_hbm.at[pl.ds(0, CHUNK)],
                              idbuf.at[b], sems.at[b]).wait()

    def wait_vals(b):
        pltpu.make_async_copy(feats_hbm.at[pl.ds(0, CHUNK)],
                              valbuf.at[b], sems.at[b + 2]).wait()

    @pl.when(wid < N)
    def _():
        zero_acc()
        for b in range(2):
            start_ids(b, wid, b)

        def pair(i, carry):
            for b in range(2):
                t = 2 * i + b
                wait_ids(b)
                scatter_slot(b, False)
                nxt = t + 2

                @pl.when(nxt < NCHUNK)
                def _():
                    start_ids(b, wid, nxt)
            return carry
        lax.fori_loop(0, NCHUNK // 2, pair, 0)
        pltpu.sync_copy(acc, cnts_hbm.at[wid])

    for k in range(CPW):
        ch = wid * CPW + k
        zero_acc()
        for b in range(2):
            n0 = b // NCHUNK
            start_ids(b, n0, b % NCHUNK)
            start_vals(b, n0 * C + ch, b % NCHUNK)

        def pair(i, carry, ch=ch):
            for b in range(2):
                t = 2 * i + b
                wait_ids(b)
                wait_vals(b)
                scatter_slot(b, True)
                nxt = t + 2

                @pl.when(nxt < TOTCH)
                def _():
                    n = lax.shift_right_logical(nxt, 3)
                    q = lax.bitwise_and(nxt, NCHUNK - 1)
                    start_ids(b, n, q)
                    start_vals(b, n * C + ch, q)
            return carry
        lax.fori_loop(0, TOTCH // 2, pair, 0)
        pltpu.sync_copy(acc, sums_hbm.at[ch])


_sc_scatter = pl.kernel(
    _sc_body,
    out_type=[
        jax.ShapeDtypeStruct((C, V), jnp.float32),
        jax.ShapeDtypeStruct((N, V), jnp.float32),
    ],
    mesh=plsc.VectorSubcoreMesh(
        core_axis_name="c", subcore_axis_name="s",
        num_cores=NC, num_subcores=NS,
    ),
    scratch_types=[
        pltpu.VMEM((V,), jnp.float32),
        pltpu.VMEM((2, CHUNK), jnp.int32),
        pltpu.VMEM((2, CHUNK), jnp.float32),
        pltpu.SemaphoreType.DMA((4,)),
    ],
    compiler_params=pltpu.CompilerParams(
        use_tc_tiling_on_sc=False, needs_layout_passes=False),
)


VB = 2048


def _tc_body(sums_ref, cnts_ref, conf_ref, w_ref, b_ref, out_ref):
    s = sums_ref[...]
    cnt = jnp.sum(cnts_ref[...], axis=0, keepdims=True)
    sv = s / jnp.maximum(cnt, 1.0)
    w = w_ref[...]
    w1 = w[:, :C]
    w2 = w[:, C:]
    a = lax.dot_general(sv, w1, (((0,), (1,)), ((), ())),
                        preferred_element_type=jnp.float32)
    b2 = lax.dot_general(conf_ref[0], w2, (((1,), (1,)), ((), ())),
                         preferred_element_type=jnp.float32)
    out_ref[0] = a + b2 + b_ref[...]


_tc_mix = pl.pallas_call(
    _tc_body,
    grid=(pl.cdiv(V, VB),),
    in_specs=[
        pl.BlockSpec((C, VB), lambda i: (0, i)),
        pl.BlockSpec((N, VB), lambda i: (0, i)),
        pl.BlockSpec((1, VB, CONF), lambda i: (0, i, 0)),
        pl.BlockSpec((OUT, C + CONF), lambda i: (0, 0)),
        pl.BlockSpec((1, OUT), lambda i: (0, 0)),
    ],
    out_specs=pl.BlockSpec((1, VB, OUT), lambda i: (0, i, 0)),
    out_shape=jax.ShapeDtypeStruct((1, V, OUT), jnp.float32),
)


def kernel(camera_pose, padded_intrinsics, padded_img_features, depths,
           padding_confidence, out_voxel_ids, W_mix, b_mix):
    feats = padded_img_features.reshape(N * C * HW)
    ids = out_voxel_ids.reshape(N * HW).astype(jnp.int32)
    sums, cnts = _sc_scatter(feats, ids)
    return _tc_mix(sums, cnts, padding_confidence, W_mix, b_mix.reshape(1, OUT))
